# baseline (speedup 1.0000x reference)
import functools

import jax
import jax.numpy as jnp
from jax import lax
from jax.experimental import pallas as pl
from jax.experimental.pallas import tpu as pltpu
from jax.experimental.pallas import tpu_sc as plsc

_N = 100000
_E = 3200000
_H = 10
_D = 20
_DP = 24


_BN = 4096


def _stage1_body(zt_ref, al_ref, ar_ref, el_ref, er_ref):
    zt = zt_ref[...]
    dn = (((0,), (0,)), ((), ()))
    el_ref[...] = lax.dot_general(zt, al_ref[...], dn,
                                  preferred_element_type=jnp.float32)
    er_ref[...] = lax.dot_general(zt, ar_ref[...], dn,
                                  preferred_element_type=jnp.float32)


def _edge_features(Zt, Al, Ar):
    grid = (_N + _BN - 1) // _BN
    return pl.pallas_call(
        _stage1_body,
        grid=(grid,),
        in_specs=[
            pl.BlockSpec((_H * _D, _BN), lambda i: (0, i)),
            pl.BlockSpec((_H * _D, _DP), lambda i: (0, 0)),
            pl.BlockSpec((_H * _D, _DP), lambda i: (0, 0)),
        ],
        out_specs=[
            pl.BlockSpec((_BN, _DP), lambda i: (i, 0)),
            pl.BlockSpec((_BN, _DP), lambda i: (i, 0)),
        ],
        out_shape=[
            jax.ShapeDtypeStruct((_N, _DP), jnp.float32),
            jax.ShapeDtypeStruct((_N, _DP), jnp.float32),
        ],
    )(Zt, Al, Ar)


_NC = 2
_NS = 16
_NW = _NC * _NS

_CH = 128
_NCHUNK = _E // _CH
_SB = 4

_CNT_HI = 784
_N_HI = 21

_VEC = 16


def _sc_gather_add(el, er, row2d, col2d):
    mesh = plsc.VectorSubcoreMesh(core_axis_name="c", subcore_axis_name="s")

    @functools.partial(
        pl.kernel,
        out_type=jax.ShapeDtypeStruct((3, _NCHUNK, 8, _CH), jnp.float32),
        mesh=mesh,
        scratch_types=[
            pltpu.VMEM((_SB, _CH), jnp.int32),
            pltpu.VMEM((_SB, _CH), jnp.int32),
            pltpu.VMEM((_SB, _CH, _DP), jnp.float32),
            pltpu.VMEM((_SB, _CH, _DP), jnp.float32),
            pltpu.VMEM((3, _SB, 8, _CH), jnp.float32),
            pltpu.VMEM((_SB, _CH), jnp.int32),
            pltpu.VMEM((_SB, _CH), jnp.int32),
            pltpu.VMEM((_SB, _CH, _DP), jnp.float32),
            pltpu.VMEM((_SB, _CH, _DP), jnp.float32),
            pltpu.VMEM((3, _SB, 8, _CH), jnp.float32),
            pltpu.SemaphoreType.DMA,
            pltpu.SemaphoreType.DMA,
            pltpu.SemaphoreType.DMA,
            pltpu.SemaphoreType.DMA,
            pltpu.SemaphoreType.DMA,
            pltpu.SemaphoreType.DMA,
            pltpu.SemaphoreType.DMA,
            pltpu.SemaphoreType.DMA,
        ],
        compiler_params=pltpu.CompilerParams(
            use_tc_tiling_on_sc=False, needs_layout_passes=False),
    )
    def k(el_hbm, er_hbm, row_hbm, col_hbm, out_hbm,
          idx1a, idx2a, buf1a, buf2a, bufta,
          idx1b, idx2b, buf1b, buf2b, buftb,
          g1a, g2a, isa, osa, g1b, g2b, isb, osb):
        wid = lax.axis_index("s") * _NC + lax.axis_index("c")
        start = wid * _CNT_HI - jnp.maximum(wid - _N_HI, 0) * 8
        n_sb = jnp.where(wid < _N_HI, _CNT_HI // _SB, _CNT_HI // _SB - 2)

        seta = (idx1a, idx2a, buf1a, buf2a, bufta, g1a, g2a, isa, osa)
        setb = (idx1b, idx2b, buf1b, buf2b, buftb, g1b, g2b, isb, osb)

        lane = lax.iota(jnp.int32, _VEC)
        u_r0, u_d, u_d3, u_s = [], [], [], []
        for u in range(5):
            o = u * _VEC + lane
            r0 = o // _D
            d = o - r0 * _D
            d3 = d // 8
            u_r0.append(r0)
            u_d.append(d)
            u_d3.append(d3)
            u_s.append(d - d3 * 8)

        def issue_idx(st, t):
            (idx1, idx2, _b1, _b2, _bt, _g1, _g2, isem, _os) = st
            c = start + t * _SB
            pltpu.async_copy(row_hbm.at[pl.ds(c, _SB), :], idx1, isem)
            pltpu.async_copy(col_hbm.at[pl.ds(c, _SB), :], idx2, isem)

        def wait_idx(st):
            (idx1, idx2, _b1, _b2, _bt, _g1, _g2, isem, _os) = st
            pltpu.make_async_copy(row_hbm.at[pl.ds(0, _SB), :], idx1,
                                  isem).wait()
            pltpu.make_async_copy(col_hbm.at[pl.ds(0, _SB), :], idx2,
                                  isem).wait()

        def fire_gathers(st):
            (idx1, idx2, b1, b2, _bt, g1, g2, _is, _os) = st
            for j in range(_SB):
                pltpu.async_copy(el_hbm.at[idx1.at[j]], b1.at[j], g1)
                pltpu.async_copy(er_hbm.at[idx2.at[j]], b2.at[j], g2)

        def wait_gathers(st):
            (idx1, idx2, b1, b2, _bt, g1, g2, _is, _os) = st
            for j in range(_SB):
                pltpu.make_async_copy(el_hbm.at[idx1.at[j]], b1.at[j],
                                      g1).wait()
                pltpu.make_async_copy(er_hbm.at[idx2.at[j]], b2.at[j],
                                      g2).wait()

        def issue_out(st, t):
            (_i1, _i2, _b1, _b2, bt, _g1, _g2, _is, osem) = st
            c = start + t * _SB
            for i in range(3):
                pltpu.async_copy(bt.at[i], out_hbm.at[i, pl.ds(c, _SB), :, :],
                                 osem)

        def wait_out(st):
            (_i1, _i2, _b1, _b2, bt, _g1, _g2, _is, osem) = st
            for i in range(3):
                pltpu.make_async_copy(
                    bt.at[i], out_hbm.at[i, pl.ds(0, _SB), :, :], osem).wait()

        def add_all(st):
            (_i1, _i2, buf1, buf2, buft, _g1, _g2, _is, _os) = st

            @plsc.parallel_loop(0, _SB * _CH * _D // (_VEC * 5), unroll=4)
            def body(g):
                roff = g * 4
                for u in range(5):
                    r = u_r0[u] + roff
                    j = r >> 7
                    rr = r & (_CH - 1)
                    a = plsc.load_gather(buf1, [j, rr, u_d[u]])
                    b = plsc.load_gather(buf2, [j, rr, u_d[u]])
                    plsc.store_scatter(buft, [u_d3[u], j, u_s[u], rr], a + b)

        def step(t, stp, stq):
            wait_gathers(stp)

            @pl.when(t + 1 < n_sb)
            def _():
                wait_idx(stq)
                fire_gathers(stq)

            @pl.when(t + 2 < n_sb)
            def _():
                issue_idx(stp, t + 2)

            @pl.when(t >= 2)
            def _():
                wait_out(stp)
            add_all(stp)
            issue_out(stp, t)

        issue_idx(seta, 0)
        wait_idx(seta)
        fire_gathers(seta)
        issue_idx(setb, 1)

        def pair_body(s2, _):
            t = s2 * 2
            step(t, seta, setb)
            step(t + 1, setb, seta)
            return 0

        lax.fori_loop(0, n_sb // 2, pair_body, 0)

        wait_out(seta)
        wait_out(setb)

    return k(el, er, row2d, col2d)


def kernel(Z, row, col, a_l, a_r):
    Zt = Z.reshape(_N, _H * _D).T
    eyep = jnp.concatenate(
        [jnp.eye(_D, dtype=jnp.float32),
         jnp.zeros((_D, _DP - _D), jnp.float32)], axis=1)
    Al = (a_l[0][:, :, None] * eyep[None]).reshape(_H * _D, _DP)
    Ar = (a_r[0][:, :, None] * eyep[None]).reshape(_H * _D, _DP)
    el, er = _edge_features(Zt, Al, Ar)

    row2d = row.astype(jnp.int32).reshape(_NCHUNK, _CH)
    col2d = col.astype(jnp.int32).reshape(_NCHUNK, _CH)
    out4d = _sc_gather_add(el, er, row2d, col2d)
    return out4d.transpose(1, 3, 0, 2).reshape(_E, 3 * 8)[:, :_D]


# speedup vs baseline: 1.2387x; 1.2387x over previous
import functools

import jax
import jax.numpy as jnp
from jax import lax
from jax.experimental import pallas as pl
from jax.experimental.pallas import tpu as pltpu
from jax.experimental.pallas import tpu_sc as plsc

_N = 100000
_E = 3200000
_H = 10
_D = 20
_DP = 24


_BN = 4096


def _stage1_body(zt_ref, al_ref, ar_ref, el_ref, er_ref):
    zt = zt_ref[...]
    dn = (((0,), (0,)), ((), ()))
    el_ref[...] = lax.dot_general(zt, al_ref[...], dn,
                                  preferred_element_type=jnp.float32)
    er_ref[...] = lax.dot_general(zt, ar_ref[...], dn,
                                  preferred_element_type=jnp.float32)


def _edge_features(Zt, Al, Ar):
    grid = (_N + _BN - 1) // _BN
    return pl.pallas_call(
        _stage1_body,
        grid=(grid,),
        in_specs=[
            pl.BlockSpec((_H * _D, _BN), lambda i: (0, i)),
            pl.BlockSpec((_H * _D, _DP), lambda i: (0, 0)),
            pl.BlockSpec((_H * _D, _DP), lambda i: (0, 0)),
        ],
        out_specs=[
            pl.BlockSpec((_BN, _DP), lambda i: (i, 0)),
            pl.BlockSpec((_BN, _DP), lambda i: (i, 0)),
        ],
        out_shape=[
            jax.ShapeDtypeStruct((_N, _DP), jnp.float32),
            jax.ShapeDtypeStruct((_N, _DP), jnp.float32),
        ],
    )(Zt, Al, Ar)


_NC = 2
_NS = 16
_NW = _NC * _NS

_CH = 128
_NCHUNK = _E // _CH
_SB = 4

_CNT_HI = 784
_N_HI = 21

_VEC = 16


def _sc_gather_add(el, er, row2d, col2d):
    mesh = plsc.VectorSubcoreMesh(core_axis_name="c", subcore_axis_name="s")

    @functools.partial(
        pl.kernel,
        out_type=jax.ShapeDtypeStruct((3, _NCHUNK, 8, _CH), jnp.float32),
        mesh=mesh,
        scratch_types=[
            pltpu.VMEM((_SB, _CH), jnp.int32),
            pltpu.VMEM((_SB, _CH), jnp.int32),
            pltpu.VMEM((_SB, _CH, _DP), jnp.float32),
            pltpu.VMEM((_SB, _CH, _DP), jnp.float32),
            pltpu.VMEM((3, _SB, 8, _CH), jnp.float32),
            pltpu.VMEM((_SB, _CH), jnp.int32),
            pltpu.VMEM((_SB, _CH), jnp.int32),
            pltpu.VMEM((_SB, _CH, _DP), jnp.float32),
            pltpu.VMEM((_SB, _CH, _DP), jnp.float32),
            pltpu.VMEM((3, _SB, 8, _CH), jnp.float32),
            pltpu.SemaphoreType.DMA,
            pltpu.SemaphoreType.DMA,
            pltpu.SemaphoreType.DMA,
            pltpu.SemaphoreType.DMA,
            pltpu.SemaphoreType.DMA,
            pltpu.SemaphoreType.DMA,
            pltpu.SemaphoreType.DMA,
            pltpu.SemaphoreType.DMA,
        ],
        compiler_params=pltpu.CompilerParams(
            use_tc_tiling_on_sc=False, needs_layout_passes=False),
    )
    def k(el_hbm, er_hbm, row_hbm, col_hbm, out_hbm,
          idx1a, idx2a, buf1a, buf2a, bufta,
          idx1b, idx2b, buf1b, buf2b, buftb,
          g1a, g2a, isa, osa, g1b, g2b, isb, osb):
        wid = lax.axis_index("s") * _NC + lax.axis_index("c")
        start = wid * _CNT_HI - jnp.maximum(wid - _N_HI, 0) * 8
        n_sb = jnp.where(wid < _N_HI, _CNT_HI // _SB, _CNT_HI // _SB - 2)

        seta = (idx1a, idx2a, buf1a, buf2a, bufta, g1a, g2a, isa, osa)
        setb = (idx1b, idx2b, buf1b, buf2b, buftb, g1b, g2b, isb, osb)

        lane = lax.iota(jnp.int32, _VEC)
        u_r0, u_d, u_d3, u_s = [], [], [], []
        for u in range(5):
            o = u * _VEC + lane
            r0 = o // _D
            d = o - r0 * _D
            d3 = d // 8
            u_r0.append(r0)
            u_d.append(d)
            u_d3.append(d3)
            u_s.append(d - d3 * 8)

        def issue_idx(st, t):
            (idx1, idx2, _b1, _b2, _bt, _g1, _g2, isem, _os) = st
            c = start + t * _SB
            pltpu.async_copy(row_hbm.at[pl.ds(c, _SB), :], idx1, isem)
            pltpu.async_copy(col_hbm.at[pl.ds(c, _SB), :], idx2, isem)

        def wait_idx(st):
            (idx1, idx2, _b1, _b2, _bt, _g1, _g2, isem, _os) = st
            pltpu.make_async_copy(row_hbm.at[pl.ds(0, _SB), :], idx1,
                                  isem).wait()
            pltpu.make_async_copy(col_hbm.at[pl.ds(0, _SB), :], idx2,
                                  isem).wait()

        def fire_gathers(st):
            (idx1, idx2, b1, b2, _bt, g1, g2, _is, _os) = st
            for j in range(_SB):
                pltpu.async_copy(el_hbm.at[idx1.at[j]], b1.at[j], g1)
                pltpu.async_copy(er_hbm.at[idx2.at[j]], b2.at[j], g2)

        def wait_gathers(st):
            (idx1, idx2, b1, b2, _bt, g1, g2, _is, _os) = st
            for j in range(_SB):
                pltpu.make_async_copy(el_hbm.at[idx1.at[j]], b1.at[j],
                                      g1).wait()
                pltpu.make_async_copy(er_hbm.at[idx2.at[j]], b2.at[j],
                                      g2).wait()

        def issue_out(st, t):
            (_i1, _i2, _b1, _b2, bt, _g1, _g2, _is, osem) = st
            c = start + t * _SB
            for i in range(3):
                pltpu.async_copy(bt.at[i], out_hbm.at[i, pl.ds(c, _SB), :, :],
                                 osem)

        def wait_out(st):
            (_i1, _i2, _b1, _b2, bt, _g1, _g2, _is, osem) = st
            for i in range(3):
                pltpu.make_async_copy(
                    bt.at[i], out_hbm.at[i, pl.ds(0, _SB), :, :], osem).wait()

        def add_all(st):
            (_i1, _i2, buf1, buf2, buft, _g1, _g2, _is, _os) = st

            @plsc.parallel_loop(0, _SB * _CH * _D // (_VEC * 5), unroll=2)
            def body(g):
                roff = g * 4
                for u in range(5):
                    r = u_r0[u] + roff
                    j = r >> 7
                    rr = r & (_CH - 1)
                    a = plsc.load_gather(buf1, [j, rr, u_d[u]])
                    b = plsc.load_gather(buf2, [j, rr, u_d[u]])
                    plsc.store_scatter(buft, [u_d3[u], j, u_s[u], rr], a + b)

        def step(t, stp, stq):
            wait_gathers(stp)

            @pl.when(t + 1 < n_sb)
            def _():
                wait_idx(stq)
                fire_gathers(stq)

            @pl.when(t + 2 < n_sb)
            def _():
                issue_idx(stp, t + 2)

            @pl.when(t >= 2)
            def _():
                wait_out(stp)
            add_all(stp)
            issue_out(stp, t)

        issue_idx(seta, 0)
        wait_idx(seta)
        fire_gathers(seta)
        issue_idx(setb, 1)

        def pair_body(s2, _):
            t = s2 * 2
            step(t, seta, setb)
            step(t + 1, setb, seta)
            return 0

        lax.fori_loop(0, n_sb // 2, pair_body, 0)

        wait_out(seta)
        wait_out(setb)

    return k(el, er, row2d, col2d)


def kernel(Z, row, col, a_l, a_r):
    Zt = Z.reshape(_N, _H * _D).T
    eyep = jnp.concatenate(
        [jnp.eye(_D, dtype=jnp.float32),
         jnp.zeros((_D, _DP - _D), jnp.float32)], axis=1)
    Al = (a_l[0][:, :, None] * eyep[None]).reshape(_H * _D, _DP)
    Ar = (a_r[0][:, :, None] * eyep[None]).reshape(_H * _D, _DP)
    el, er = _edge_features(Zt, Al, Ar)

    row2d = row.astype(jnp.int32).reshape(_NCHUNK, _CH)
    col2d = col.astype(jnp.int32).reshape(_NCHUNK, _CH)
    out4d = _sc_gather_add(el, er, row2d, col2d)
    return out4d.transpose(1, 3, 0, 2).reshape(_E, 3 * 8)[:, :_D]
